# baseline (speedup 1.0000x reference)
import functools

import jax
import jax.numpy as jnp
from jax import lax
from jax.experimental import pallas as pl
from jax.experimental.pallas import tpu as pltpu
from jax.experimental.pallas import tpu_sc as plsc

N = 16384
B = 1024
G = 32
RB = 64
NUM_BLOCKS = N // RB
NPASS = NUM_BLOCKS // 32
BND_PAD = NUM_BLOCKS + 1 + 15


def _spmm_kernel(xT, vals, rows, cols, bounds, out, gath, acc, colv, rowv,
                 valv, offv, bndv, sem):
    nc = 2
    ns = 16
    L = 16
    c = lax.axis_index("c")
    s = lax.axis_index("s")
    wid = s * nc + c

    pltpu.sync_copy(bounds, bndv)

    def pass_body(p, _):
        blk = p * (nc * ns) + wid
        base_row = blk * RB
        bvec = bndv[pl.ds(blk, L)]
        bs = bvec[0]
        be = bvec[1]

        def z_body(i, _):
            r = i // (B // L)
            t = i % (B // L)
            acc[r, pl.ds(t * L, L)] = jnp.zeros((L,), jnp.float32)
            return 0
        lax.fori_loop(0, (RB * B) // L, z_body, 0)

        k0base = (bs // 8) * 8
        nchunks = (be - k0base + G - 1) // G

        def chunk_body(ci, _):
            k0 = k0base + ci * G
            pltpu.sync_copy(cols.at[pl.ds(k0, G)], colv)
            pltpu.sync_copy(rows.at[pl.ds(k0, G)], rowv)
            pltpu.sync_copy(vals.at[pl.ds(k0, G)], valv.at[pl.ds(0, G)])
            for g in range(G // L):
                kv = k0 + g * L + lax.iota(jnp.int32, L)
                inr = (kv >= bs) & (kv < be)
                v = valv[pl.ds(g * L, L)]
                valv[pl.ds(g * L, L)] = jnp.where(
                    inr, v, jnp.zeros((L,), jnp.float32))
                r = rowv[pl.ds(g * L, L)]
                offv[pl.ds(g * L, L)] = jnp.clip(
                    r - base_row, 0, RB - 1)
            pltpu.async_copy(xT.at[colv], gath, sem).wait()
            def acc_body(j, _):
                vj = valv[pl.ds(j, L)][0]
                off = offv[pl.ds(j, L)][0]
                for t in range(B // L):
                    acc[off, pl.ds(t * L, L)] = (
                        acc[off, pl.ds(t * L, L)]
                        + gath[j, pl.ds(t * L, L)] * vj)
                return 0
            lax.fori_loop(0, G, acc_body, 0)
            return 0
        lax.fori_loop(0, nchunks, chunk_body, 0)

        pltpu.sync_copy(acc, out.at[pl.ds(base_row, RB)])
        return 0

    lax.fori_loop(0, NPASS, pass_body, 0)


@jax.jit
def _spmm(xT, vals, rows, cols, bounds):
    mesh = plsc.VectorSubcoreMesh(core_axis_name="c", subcore_axis_name="s")
    kfn = functools.partial(
        pl.kernel,
        mesh=mesh,
        out_type=jax.ShapeDtypeStruct((N, B), jnp.float32),
        scratch_types=[
            pltpu.VMEM((G, B), jnp.float32),
            pltpu.VMEM((RB, B), jnp.float32),
            pltpu.VMEM((G,), jnp.int32),
            pltpu.VMEM((G,), jnp.int32),
            pltpu.VMEM((G + 16,), jnp.float32),
            pltpu.VMEM((G + 16,), jnp.int32),
            pltpu.VMEM((BND_PAD,), jnp.int32),
            pltpu.SemaphoreType.DMA,
        ],
    )(_spmm_kernel)
    return kfn(xT, vals, rows, cols, bounds)


def kernel(x_affine, W_vals, W_row, W_col):
    xT = x_affine.T
    rows = W_row.astype(jnp.int32)
    cols = W_col.astype(jnp.int32)
    order = jnp.argsort(rows)
    rows_s = rows[order]
    cols_s = cols[order]
    vals_s = W_vals[order]
    nnz = rows_s.shape[0]
    npad = ((nnz + G + 7) // 8) * 8
    pad = npad - nnz
    rows_s = jnp.concatenate([rows_s, jnp.full((pad,), N - 1, jnp.int32)])
    cols_s = jnp.concatenate([cols_s, jnp.zeros((pad,), jnp.int32)])
    vals_s = jnp.concatenate([vals_s, jnp.zeros((pad,), jnp.float32)])
    edges = jnp.arange(0, N + 1, RB, dtype=jnp.int32)
    bounds = jnp.searchsorted(rows_s[:nnz], edges).astype(jnp.int32)
    bounds = jnp.concatenate(
        [bounds, jnp.zeros((BND_PAD - NUM_BLOCKS - 1,), jnp.int32)])
    out_T = _spmm(xT, vals_s, rows_s, cols_s, bounds)
    return out_T.T


# speedup vs baseline: 2.3134x; 2.3134x over previous
import functools

import jax
import jax.numpy as jnp
from jax import lax
from jax.experimental import pallas as pl
from jax.experimental.pallas import tpu as pltpu
from jax.experimental.pallas import tpu_sc as plsc

N = 16384
B = 1024
G = 32
RB = 64
NUM_BLOCKS = N // RB
NPASS = NUM_BLOCKS // 32
BND_PAD = NUM_BLOCKS + 1 + 15


def _spmm_kernel(xT, vals, rows, cols, bounds, out, gath, acc, colv, rowv,
                 valv, offv, bndv, sem):
    nc = 2
    ns = 16
    L = 16
    c = lax.axis_index("c")
    s = lax.axis_index("s")
    wid = s * nc + c

    pltpu.sync_copy(bounds, bndv)

    def pass_body(p, _):
        blk = p * (nc * ns) + wid
        base_row = blk * RB
        bvec = bndv[pl.ds(blk, L)]
        bs = bvec[0]
        be = bvec[1]

        def z_body(i, _):
            r = i // (B // L)
            t = i % (B // L)
            acc[r, pl.ds(t * L, L)] = jnp.zeros((L,), jnp.float32)
            return 0
        lax.fori_loop(0, (RB * B) // L, z_body, 0)

        k0base = (bs // 8) * 8
        nchunks = (be - k0base + G - 1) // G

        def chunk_body(ci, _):
            k0 = k0base + ci * G
            pltpu.sync_copy(cols.at[pl.ds(k0, G)], colv)
            pltpu.sync_copy(rows.at[pl.ds(k0, G)], rowv)
            pltpu.sync_copy(vals.at[pl.ds(k0, G)], valv.at[pl.ds(0, G)])
            for g in range(G // L):
                kv = k0 + g * L + lax.iota(jnp.int32, L)
                inr = (kv >= bs) & (kv < be)
                v = valv[pl.ds(g * L, L)]
                valv[pl.ds(g * L, L)] = jnp.where(
                    inr, v, jnp.zeros((L,), jnp.float32))
                r = rowv[pl.ds(g * L, L)]
                offv[pl.ds(g * L, L)] = jnp.clip(
                    r - base_row, 0, RB - 1)
            pltpu.async_copy(xT.at[colv], gath, sem).wait()
            def acc_body(j, _):
                vj = valv[pl.ds(j, L)][0]
                off = offv[pl.ds(j, L)][0]
                for t in range(B // L):
                    acc[off, pl.ds(t * L, L)] = (
                        acc[off, pl.ds(t * L, L)]
                        + gath[j, pl.ds(t * L, L)] * vj)
                return 0
            lax.fori_loop(0, G, acc_body, 0)
            return 0
        lax.fori_loop(0, nchunks, chunk_body, 0)

        pltpu.sync_copy(acc, out.at[pl.ds(base_row, RB)])
        return 0

    lax.fori_loop(0, NPASS, pass_body, 0)


@jax.jit
def _spmm(xT, vals, rows, cols, bounds):
    mesh = plsc.VectorSubcoreMesh(core_axis_name="c", subcore_axis_name="s")
    kfn = functools.partial(
        pl.kernel,
        mesh=mesh,
        out_type=jax.ShapeDtypeStruct((N, B), jnp.float32),
        scratch_types=[
            pltpu.VMEM((G, B), jnp.float32),
            pltpu.VMEM((RB, B), jnp.float32),
            pltpu.VMEM((G,), jnp.int32),
            pltpu.VMEM((G,), jnp.int32),
            pltpu.VMEM((G + 16,), jnp.float32),
            pltpu.VMEM((G + 16,), jnp.int32),
            pltpu.VMEM((BND_PAD,), jnp.int32),
            pltpu.SemaphoreType.DMA,
        ],
    )(_spmm_kernel)
    return kfn(xT, vals, rows, cols, bounds)


def kernel(x_affine, W_vals, W_row, W_col):
    xT = x_affine.T
    rows = W_row.astype(jnp.int32)
    cols = W_col.astype(jnp.int32)
    rows_s, cols_s, vals_s = lax.sort((rows, cols, W_vals), num_keys=1)
    nnz = rows_s.shape[0]
    npad = ((nnz + G + 7) // 8) * 8
    pad = npad - nnz
    rows_s = jnp.concatenate([rows_s, jnp.full((pad,), N - 1, jnp.int32)])
    cols_s = jnp.concatenate([cols_s, jnp.zeros((pad,), jnp.int32)])
    vals_s = jnp.concatenate([vals_s, jnp.zeros((pad,), jnp.float32)])
    edges = jnp.arange(0, N + 1, RB, dtype=jnp.int32)
    bounds = jnp.searchsorted(rows_s[:nnz], edges).astype(jnp.int32)
    bounds = jnp.concatenate(
        [bounds, jnp.zeros((BND_PAD - NUM_BLOCKS - 1,), jnp.int32)])
    out_T = _spmm(xT, vals_s, rows_s, cols_s, bounds)
    return out_T.T


# speedup vs baseline: 4.4762x; 1.9349x over previous
import functools

import jax
import jax.numpy as jnp
from jax import lax
from jax.experimental import pallas as pl
from jax.experimental.pallas import tpu as pltpu
from jax.experimental.pallas import tpu_sc as plsc

N = 16384
B = 1024
G = 32
RB = 64
NUM_BLOCKS = N // RB
NPASS = NUM_BLOCKS // 32
BND_PAD = NUM_BLOCKS + 1 + 15
L = 16
NSTRIP = 4
SCOLS = B // NSTRIP
SWORDS = SCOLS // 32


def _strip_accum(gath, acc, valv, offv, bset, sp):
    zero16 = jnp.zeros((L,), jnp.float32)
    regs0 = (zero16,) * (2 * SWORDS)
    off0 = offv[bset, pl.ds(0, L)][0]

    def flush(off, regs):
        for w in range(SWORDS):
            cbase = sp * SCOLS + w * 32
            acc[off, pl.ds(cbase, L)] = acc[off, pl.ds(cbase, L)] + regs[2 * w]
            acc[off, pl.ds(cbase + L, L)] = (
                acc[off, pl.ds(cbase + L, L)] + regs[2 * w + 1])

    def j_body(j, carry):
        cur_off = carry[0]
        regs = carry[1:]
        vj = valv[bset, pl.ds(j, L)][0]
        off = offv[bset, pl.ds(j, L)][0]

        def do_flush(r):
            flush(cur_off, r)
            return regs0
        regs = lax.cond(off != cur_off, do_flush, lambda r: r, regs)

        new = []
        for w in range(SWORDS):
            wv = gath[j, pl.ds(sp * (SCOLS // 2) + w * L, L)]
            f_lo = lax.bitcast_convert_type(wv << 16, jnp.float32)
            f_hi = lax.bitcast_convert_type(wv & jnp.int32(-65536), jnp.float32)
            new.append(regs[2 * w] + f_lo * vj)
            new.append(regs[2 * w + 1] + f_hi * vj)
        return (off,) + tuple(new)

    final = lax.fori_loop(0, G, j_body, (off0,) + regs0)
    flush(final[0], final[1:])


def _spmm_kernel(xT, vals, rows, cols, bounds, out, gb0, gb1, acc, colv, rowv,
                 valv, offv, bndv, sem0, sem1):
    nc = 2
    ns = 16
    c = lax.axis_index("c")
    s = lax.axis_index("s")
    wid = s * nc + c

    gbs = (gb0, gb1)
    sems = (sem0, sem1)

    pltpu.sync_copy(bounds, bndv)

    def load_idx(bset, k0):
        pltpu.sync_copy(cols.at[pl.ds(k0, G)], colv.at[bset])
        pltpu.sync_copy(rows.at[pl.ds(k0, G)], rowv.at[bset])
        pltpu.sync_copy(vals.at[pl.ds(k0, G)],
                        valv.at[bset, pl.ds(0, G)])

    def prep_masks(bset, k0, bs, be, base_row):
        for g in range(G // L):
            kv = k0 + g * L + lax.iota(jnp.int32, L)
            inr = (kv >= bs) & (kv < be)
            v = valv[bset, pl.ds(g * L, L)]
            valv[bset, pl.ds(g * L, L)] = jnp.where(
                inr, v, jnp.zeros((L,), jnp.float32))
            r = rowv[bset, pl.ds(g * L, L)]
            offv[bset, pl.ds(g * L, L)] = jnp.clip(
                r - base_row, 0, RB - 1)

    def start_gather(bset):
        pltpu.async_copy(xT.at[colv.at[bset]], gbs[bset], sems[bset])

    def wait_gather(bset):
        pltpu.make_async_copy(xT.at[colv.at[bset]], gbs[bset],
                              sems[bset]).wait()

    def pass_body(p, _):
        blk = p * (nc * ns) + wid
        base_row = blk * RB
        bvec = bndv[pl.ds(blk, L)]
        bs = bvec[0]
        be = bvec[1]

        def z_body(i, _):
            r = i // (B // L)
            t = i % (B // L)
            acc[r, pl.ds(t * L, L)] = jnp.zeros((L,), jnp.float32)
            return 0
        lax.fori_loop(0, (RB * B) // L, z_body, 0)

        k0base = (bs // 8) * 8
        nchunks = (be - k0base + G - 1) // G

        @pl.when(nchunks > 0)
        def _prologue():
            load_idx(0, k0base)
            prep_masks(0, k0base, bs, be, base_row)
            start_gather(0)

        def process(ci, bset):
            k0 = k0base + ci * G

            @pl.when(ci + 1 < nchunks)
            def _next():
                load_idx(1 - bset, k0 + G)
                prep_masks(1 - bset, k0 + G, bs, be, base_row)
                start_gather(1 - bset)
            wait_gather(bset)
            for sp in range(NSTRIP):
                _strip_accum(gbs[bset], acc, valv, offv, bset, sp)

        def pair_body(k, _):
            ci0 = 2 * k
            process(ci0, 0)

            @pl.when(ci0 + 1 < nchunks)
            def _odd():
                process(ci0 + 1, 1)
            return 0
        lax.fori_loop(0, (nchunks + 1) // 2, pair_body, 0)

        pltpu.sync_copy(acc, out.at[pl.ds(base_row, RB)])
        return 0

    lax.fori_loop(0, NPASS, pass_body, 0)


@jax.jit
def _spmm(xT, vals, rows, cols, bounds):
    mesh = plsc.VectorSubcoreMesh(core_axis_name="c", subcore_axis_name="s")
    kfn = functools.partial(
        pl.kernel,
        mesh=mesh,
        out_type=jax.ShapeDtypeStruct((N, B), jnp.float32),
        scratch_types=[
            pltpu.VMEM((G, B // 2), jnp.int32),
            pltpu.VMEM((G, B // 2), jnp.int32),
            pltpu.VMEM((RB, B), jnp.float32),
            pltpu.VMEM((2, G), jnp.int32),
            pltpu.VMEM((2, G), jnp.int32),
            pltpu.VMEM((2, G + 16), jnp.float32),
            pltpu.VMEM((2, G + 16), jnp.int32),
            pltpu.VMEM((BND_PAD,), jnp.int32),
            pltpu.SemaphoreType.DMA,
            pltpu.SemaphoreType.DMA,
        ],
    )(_spmm_kernel)
    return kfn(xT, vals, rows, cols, bounds)


def kernel(x_affine, W_vals, W_row, W_col):
    xT = x_affine.T
    xTb = (xT.astype(jnp.bfloat16)
           .reshape(N, B // 32, 2, L)
           .transpose(0, 1, 3, 2)
           .reshape(N, B // 2, 2))
    xTb = lax.bitcast_convert_type(xTb, jnp.int32)
    rows = W_row.astype(jnp.int32)
    cols = W_col.astype(jnp.int32)
    rows_s, cols_s, vals_s = lax.sort((rows, cols, W_vals), num_keys=1)
    nnz = rows_s.shape[0]
    npad = ((nnz + G + 7) // 8) * 8
    pad = npad - nnz
    rows_s = jnp.concatenate([rows_s, jnp.full((pad,), N - 1, jnp.int32)])
    cols_s = jnp.concatenate([cols_s, jnp.zeros((pad,), jnp.int32)])
    vals_s = jnp.concatenate([vals_s, jnp.zeros((pad,), jnp.float32)])
    edges = jnp.arange(0, N + 1, RB, dtype=jnp.int32)
    bounds = jnp.searchsorted(rows_s[:nnz], edges).astype(jnp.int32)
    bounds = jnp.concatenate(
        [bounds, jnp.zeros((BND_PAD - NUM_BLOCKS - 1,), jnp.int32)])
    out_T = _spmm(xTb, vals_s, rows_s, cols_s, bounds)
    return out_T.T


# speedup vs baseline: 4.8748x; 1.0890x over previous
import functools

import jax
import jax.numpy as jnp
from jax import lax
from jax.experimental import pallas as pl
from jax.experimental.pallas import tpu as pltpu
from jax.experimental.pallas import tpu_sc as plsc

N = 16384
B = 1024
G = 32
RB = 64
NUM_BLOCKS = N // RB
NPASS = NUM_BLOCKS // 32
BND_PAD = NUM_BLOCKS + 1 + 15
L = 16
NSTRIP = 4
SCOLS = B // NSTRIP
SWORDS = SCOLS // 32
BK = 512
BCH = BK // G


def _strip_accum(gath, acc, vbuf, offv, jbase, sp):
    zero16 = jnp.zeros((L,), jnp.float32)
    regs0 = (zero16,) * (2 * SWORDS)
    off0 = offv[pl.ds(jbase, L)][0]

    def flush(off, regs):
        for w in range(SWORDS):
            cbase = sp * SCOLS + w * 32
            acc[off, pl.ds(cbase, L)] = acc[off, pl.ds(cbase, L)] + regs[2 * w]
            acc[off, pl.ds(cbase + L, L)] = (
                acc[off, pl.ds(cbase + L, L)] + regs[2 * w + 1])

    def j_body(j, carry):
        cur_off = carry[0]
        regs = carry[1:]
        vj = vbuf[pl.ds((jbase + j) * L, L)]
        off = offv[pl.ds(jbase + j, L)][0]

        def do_flush(r):
            flush(cur_off, r)
            return regs0
        regs = lax.cond(off != cur_off, do_flush, lambda r: r, regs)

        new = []
        for w in range(SWORDS):
            wv = gath[j, pl.ds(sp * (SCOLS // 2) + w * L, L)]
            f_lo = lax.bitcast_convert_type(wv << 16, jnp.float32)
            f_hi = lax.bitcast_convert_type(wv & jnp.int32(-65536), jnp.float32)
            new.append(regs[2 * w] + f_lo * vj)
            new.append(regs[2 * w + 1] + f_hi * vj)
        return (off,) + tuple(new)

    final = lax.fori_loop(0, G, j_body, (off0,) + regs0)
    flush(final[0], final[1:])


def _spmm_kernel(xT, vals, rows, cols, bounds, out, gb0, gb1, acc, colv, rowv,
                 valv, offv, vbuf, bndv, sem0, sem1):
    nc = 2
    ns = 16
    c = lax.axis_index("c")
    s = lax.axis_index("s")
    wid = s * nc + c

    gbs = (gb0, gb1)
    sems = (sem0, sem1)

    pltpu.sync_copy(bounds, bndv)

    def start_gather(bset, jbase):
        pltpu.async_copy(xT.at[colv.at[pl.ds(jbase, G)]], gbs[bset],
                         sems[bset])

    def wait_gather(bset, jbase):
        pltpu.make_async_copy(xT.at[colv.at[pl.ds(jbase, G)]], gbs[bset],
                              sems[bset]).wait()

    def pass_body(p, _):
        blk = p * (nc * ns) + wid
        base_row = blk * RB
        bvec = bndv[pl.ds(blk, L)]
        bs = bvec[0]
        be = bvec[1]

        def z_body(i, _):
            r = i // (B // L)
            t = i % (B // L)
            acc[r, pl.ds(t * L, L)] = jnp.zeros((L,), jnp.float32)
            return 0
        lax.fori_loop(0, (RB * B) // L, z_body, 0)

        k0base = (bs // 8) * 8
        total = be - k0base
        nbatches = (total + BK - 1) // BK

        def batch_body(bi, _):
            bk0 = k0base + bi * BK
            pltpu.sync_copy(cols.at[pl.ds(bk0, BK)], colv.at[pl.ds(0, BK)])
            pltpu.sync_copy(rows.at[pl.ds(bk0, BK)], rowv)
            pltpu.sync_copy(vals.at[pl.ds(bk0, BK)], valv.at[pl.ds(0, BK)])
            for g in range(BK // L):
                kv = bk0 + g * L + lax.iota(jnp.int32, L)
                inr = (kv >= bs) & (kv < be)
                v = valv[pl.ds(g * L, L)]
                valv[pl.ds(g * L, L)] = jnp.where(
                    inr, v, jnp.zeros((L,), jnp.float32))
                r = rowv[pl.ds(g * L, L)]
                offv[pl.ds(g * L, L)] = jnp.clip(
                    r - base_row, 0, RB - 1)
            def vb_body(j, _):
                vj = valv[pl.ds(j, L)][0]
                vbuf[pl.ds(j * L, L)] = jnp.full((L,), vj, jnp.float32)
                return 0
            lax.fori_loop(0, BK, vb_body, 0)

            nchunks = jnp.minimum((be - bk0 + G - 1) // G, BCH)

            @pl.when(nchunks > 0)
            def _prologue():
                start_gather(0, 0)

            def process(ci, bset):
                @pl.when(ci + 1 < nchunks)
                def _next():
                    start_gather(1 - bset, (ci + 1) * G)
                wait_gather(bset, ci * G)
                for sp in range(NSTRIP):
                    _strip_accum(gbs[bset], acc, vbuf, offv, ci * G, sp)

            def pair_body(k, _):
                ci0 = 2 * k
                process(ci0, 0)

                @pl.when(ci0 + 1 < nchunks)
                def _odd():
                    process(ci0 + 1, 1)
                return 0
            lax.fori_loop(0, (nchunks + 1) // 2, pair_body, 0)
            return 0
        lax.fori_loop(0, nbatches, batch_body, 0)

        pltpu.sync_copy(acc, out.at[pl.ds(base_row, RB)])
        return 0

    lax.fori_loop(0, NPASS, pass_body, 0)


@jax.jit
def _spmm(xT, vals, rows, cols, bounds):
    mesh = plsc.VectorSubcoreMesh(core_axis_name="c", subcore_axis_name="s")
    kfn = functools.partial(
        pl.kernel,
        mesh=mesh,
        out_type=jax.ShapeDtypeStruct((N, B), jnp.float32),
        scratch_types=[
            pltpu.VMEM((G, B // 2), jnp.int32),
            pltpu.VMEM((G, B // 2), jnp.int32),
            pltpu.VMEM((RB, B), jnp.float32),
            pltpu.VMEM((BK + 16,), jnp.int32),
            pltpu.VMEM((BK,), jnp.int32),
            pltpu.VMEM((BK + 16,), jnp.float32),
            pltpu.VMEM((BK + 16,), jnp.int32),
            pltpu.VMEM((BK * L,), jnp.float32),
            pltpu.VMEM((BND_PAD,), jnp.int32),
            pltpu.SemaphoreType.DMA,
            pltpu.SemaphoreType.DMA,
        ],
    )(_spmm_kernel)
    return kfn(xT, vals, rows, cols, bounds)


def kernel(x_affine, W_vals, W_row, W_col):
    xT = x_affine.T
    xTb = (xT.astype(jnp.bfloat16)
           .reshape(N, B // 32, 2, L)
           .transpose(0, 1, 3, 2)
           .reshape(N, B // 2, 2))
    xTb = lax.bitcast_convert_type(xTb, jnp.int32)
    rows = W_row.astype(jnp.int32)
    cols = W_col.astype(jnp.int32)
    rows_s, cols_s, vals_s = lax.sort((rows, cols, W_vals), num_keys=1)
    nnz = rows_s.shape[0]
    npad = ((nnz + 512 + 64) // 8) * 8
    pad = npad - nnz
    rows_s = jnp.concatenate([rows_s, jnp.full((pad,), N - 1, jnp.int32)])
    cols_s = jnp.concatenate([cols_s, jnp.zeros((pad,), jnp.int32)])
    vals_s = jnp.concatenate([vals_s, jnp.zeros((pad,), jnp.float32)])
    edges = jnp.arange(0, N + 1, RB, dtype=jnp.int32)
    bounds = jnp.searchsorted(rows_s[:nnz], edges).astype(jnp.int32)
    bounds = jnp.concatenate(
        [bounds, jnp.zeros((BND_PAD - NUM_BLOCKS - 1,), jnp.int32)])
    out_T = _spmm(xTb, vals_s, rows_s, cols_s, bounds)
    return out_T.T


# speedup vs baseline: 6.6633x; 1.3669x over previous
import functools

import jax
import jax.numpy as jnp
from jax import lax
from jax.experimental import pallas as pl
from jax.experimental.pallas import tpu as pltpu
from jax.experimental.pallas import tpu_sc as plsc

N = 16384
B = 1024
G = 32
RB = 64
NUM_BLOCKS = N // RB
NPASS = NUM_BLOCKS // 32
BND_PAD = NUM_BLOCKS + 1 + 15
L = 16
NSTRIP = 2
SCOLS = B // NSTRIP
SWORDS = SCOLS // 32
BK = 512
BCH = BK // G


def _strip_accum(gath, acc, vbuf, offv, jbase, sp):
    zero16 = jnp.zeros((L,), jnp.float32)
    regs0 = (zero16,) * (2 * SWORDS)
    off0 = offv[pl.ds(jbase, L)][0]

    def flush(off, regs):
        for w in range(SWORDS):
            cbase = sp * SCOLS + w * 32
            acc[off, pl.ds(cbase, L)] = acc[off, pl.ds(cbase, L)] + regs[2 * w]
            acc[off, pl.ds(cbase + L, L)] = (
                acc[off, pl.ds(cbase + L, L)] + regs[2 * w + 1])

    def j_body(j, carry):
        cur_off = carry[0]
        regs = carry[1:]
        vj = vbuf[pl.ds((jbase + j) * L, L)]
        off = offv[pl.ds(jbase + j, L)][0]

        def do_flush(r):
            flush(cur_off, r)
            return regs0
        regs = lax.cond(off != cur_off, do_flush, lambda r: r, regs)

        new = []
        for w in range(SWORDS):
            wv = gath[j, pl.ds(sp * (SCOLS // 2) + w * L, L)]
            f_lo = lax.bitcast_convert_type(wv << 16, jnp.float32)
            f_hi = lax.bitcast_convert_type(wv & jnp.int32(-65536), jnp.float32)
            new.append(regs[2 * w] + f_lo * vj)
            new.append(regs[2 * w + 1] + f_hi * vj)
        return (off,) + tuple(new)

    final = lax.fori_loop(0, G, j_body, (off0,) + regs0)
    flush(final[0], final[1:])


def _spmm_kernel(xT, vals, rows, cols, bounds, out, gb0, gb1, acc, colv, rowv,
                 valv, offv, vbuf, bndv, sem0, sem1):
    nc = 2
    ns = 16
    c = lax.axis_index("c")
    s = lax.axis_index("s")
    wid = s * nc + c

    gbs = (gb0, gb1)
    sems = (sem0, sem1)

    pltpu.sync_copy(bounds, bndv)

    def start_gather(bset, jbase):
        pltpu.async_copy(xT.at[colv.at[pl.ds(jbase, G)]], gbs[bset],
                         sems[bset])

    def wait_gather(bset, jbase):
        pltpu.make_async_copy(xT.at[colv.at[pl.ds(jbase, G)]], gbs[bset],
                              sems[bset]).wait()

    def pass_body(p, _):
        blk = p * (nc * ns) + wid
        base_row = blk * RB
        bvec = bndv[pl.ds(blk, L)]
        bs = bvec[0]
        be = bvec[1]

        z16 = jnp.zeros((L,), jnp.float32)

        def z_body(r, _):
            for t in range(B // L):
                acc[r, pl.ds(t * L, L)] = z16
            return 0
        lax.fori_loop(0, RB, z_body, 0)

        k0base = (bs // 8) * 8
        total = be - k0base
        nbatches = (total + BK - 1) // BK

        def batch_body(bi, _):
            bk0 = k0base + bi * BK
            pltpu.sync_copy(cols.at[pl.ds(bk0, BK)], colv.at[pl.ds(0, BK)])
            pltpu.sync_copy(rows.at[pl.ds(bk0, BK)], rowv)
            pltpu.sync_copy(vals.at[pl.ds(bk0, BK)], valv.at[pl.ds(0, BK)])
            for g in range(BK // L):
                kv = bk0 + g * L + lax.iota(jnp.int32, L)
                inr = (kv >= bs) & (kv < be)
                v = valv[pl.ds(g * L, L)]
                valv[pl.ds(g * L, L)] = jnp.where(
                    inr, v, jnp.zeros((L,), jnp.float32))
                r = rowv[pl.ds(g * L, L)]
                offv[pl.ds(g * L, L)] = jnp.clip(
                    r - base_row, 0, RB - 1)
            def vb_body(j, _):
                vj = valv[pl.ds(j, L)][0]
                vbuf[pl.ds(j * L, L)] = jnp.full((L,), vj, jnp.float32)
                return 0
            lax.fori_loop(0, BK, vb_body, 0)

            nchunks = jnp.minimum((be - bk0 + G - 1) // G, BCH)

            @pl.when(nchunks > 0)
            def _prologue():
                start_gather(0, 0)

            def process(ci, bset):
                @pl.when(ci + 1 < nchunks)
                def _next():
                    start_gather(1 - bset, (ci + 1) * G)
                wait_gather(bset, ci * G)
                for sp in range(NSTRIP):
                    _strip_accum(gbs[bset], acc, vbuf, offv, ci * G, sp)

            def pair_body(k, _):
                ci0 = 2 * k
                process(ci0, 0)

                @pl.when(ci0 + 1 < nchunks)
                def _odd():
                    process(ci0 + 1, 1)
                return 0
            lax.fori_loop(0, (nchunks + 1) // 2, pair_body, 0)
            return 0
        lax.fori_loop(0, nbatches, batch_body, 0)

        pltpu.sync_copy(acc, out.at[pl.ds(base_row, RB)])
        return 0

    lax.fori_loop(0, NPASS, pass_body, 0)


@jax.jit
def _spmm(xT, vals, rows, cols, bounds):
    mesh = plsc.VectorSubcoreMesh(core_axis_name="c", subcore_axis_name="s")
    kfn = functools.partial(
        pl.kernel,
        mesh=mesh,
        out_type=jax.ShapeDtypeStruct((N, B), jnp.float32),
        scratch_types=[
            pltpu.VMEM((G, B // 2), jnp.int32),
            pltpu.VMEM((G, B // 2), jnp.int32),
            pltpu.VMEM((RB, B), jnp.float32),
            pltpu.VMEM((BK + 16,), jnp.int32),
            pltpu.VMEM((BK,), jnp.int32),
            pltpu.VMEM((BK + 16,), jnp.float32),
            pltpu.VMEM((BK + 16,), jnp.int32),
            pltpu.VMEM((BK * L,), jnp.float32),
            pltpu.VMEM((BND_PAD,), jnp.int32),
            pltpu.SemaphoreType.DMA,
            pltpu.SemaphoreType.DMA,
        ],
    )(_spmm_kernel)
    return kfn(xT, vals, rows, cols, bounds)


def kernel(x_affine, W_vals, W_row, W_col):
    xT = x_affine.T
    xTb = (xT.astype(jnp.bfloat16)
           .reshape(N, B // 32, 2, L)
           .transpose(0, 1, 3, 2)
           .reshape(N, B // 2, 2))
    xTb = lax.bitcast_convert_type(xTb, jnp.int32)
    rows = W_row.astype(jnp.int32)
    cols = W_col.astype(jnp.int32)
    rows_s, cols_s, vals_s = lax.sort((rows, cols, W_vals), num_keys=1)
    nnz = rows_s.shape[0]
    npad = ((nnz + 512 + 64) // 8) * 8
    pad = npad - nnz
    rows_s = jnp.concatenate([rows_s, jnp.full((pad,), N - 1, jnp.int32)])
    cols_s = jnp.concatenate([cols_s, jnp.zeros((pad,), jnp.int32)])
    vals_s = jnp.concatenate([vals_s, jnp.zeros((pad,), jnp.float32)])
    edges = jnp.arange(0, N + 1, RB, dtype=jnp.int32)
    bounds = jnp.searchsorted(rows_s[:nnz], edges).astype(jnp.int32)
    bounds = jnp.concatenate(
        [bounds, jnp.zeros((BND_PAD - NUM_BLOCKS - 1,), jnp.int32)])
    out_T = _spmm(xTb, vals_s, rows_s, cols_s, bounds)
    return out_T.T


# speedup vs baseline: 7.9559x; 1.1940x over previous
import functools

import jax
import jax.numpy as jnp
from jax import lax
from jax.experimental import pallas as pl
from jax.experimental.pallas import tpu as pltpu
from jax.experimental.pallas import tpu_sc as plsc

N = 16384
B = 1024
G = 32
RB = 64
NUM_BLOCKS = N // RB
NPASS = NUM_BLOCKS // 32
BND_PAD = NUM_BLOCKS + 1 + 15
L = 16
NSTRIP = 2
SCOLS = B // NSTRIP
SWORDS = SCOLS // 32
BK = 512
BCH = BK // G


def _seg_scan(offv, segb, jbase):
    def sc_body(j, carry):
        cnt, prev = carry
        off = offv[pl.ds(jbase + j, L)][0]
        changed = off != prev

        @pl.when(changed)
        def _mark():
            segb[cnt] = j
        return (cnt + jnp.where(changed, 1, 0), off)

    cnt, _ = lax.fori_loop(0, G, sc_body, (0, jnp.int32(-1)))
    segb[cnt] = G
    return cnt


def _seg_accum(gath, acc, vbuf, offv, segb, jbase, nseg, sp):
    zero16 = jnp.zeros((L,), jnp.float32)
    regs0 = (zero16,) * (2 * SWORDS)

    def s_body(si, _):
        jstart = segb[si]
        jend = segb[si + 1]
        off = offv[pl.ds(jbase + jstart, L)][0]

        def j_body(j, regs):
            vj = vbuf[pl.ds((jbase + j) * L, L)]
            new = []
            for w in range(SWORDS):
                wv = gath[j, pl.ds(sp * (SCOLS // 2) + w * L, L)]
                f_lo = lax.bitcast_convert_type(wv << 16, jnp.float32)
                f_hi = lax.bitcast_convert_type(
                    wv & jnp.int32(-65536), jnp.float32)
                new.append(regs[2 * w] + f_lo * vj)
                new.append(regs[2 * w + 1] + f_hi * vj)
            return tuple(new)

        regs = lax.fori_loop(jstart, jend, j_body, regs0)
        for w in range(SWORDS):
            cbase = sp * SCOLS + w * 32
            acc[off, pl.ds(cbase, L)] = (
                acc[off, pl.ds(cbase, L)] + regs[2 * w])
            acc[off, pl.ds(cbase + L, L)] = (
                acc[off, pl.ds(cbase + L, L)] + regs[2 * w + 1])
        return 0

    lax.fori_loop(0, nseg, s_body, 0)


def _spmm_kernel(xT, vals, rows, cols, bounds, out, gb0, gb1, acc, colv, rowv,
                 valv, offv, vbuf, bndv, segb, sem0, sem1):
    nc = 2
    ns = 16
    c = lax.axis_index("c")
    s = lax.axis_index("s")
    wid = s * nc + c

    gbs = (gb0, gb1)
    sems = (sem0, sem1)

    pltpu.sync_copy(bounds, bndv)

    def start_gather(bset, jbase):
        pltpu.async_copy(xT.at[colv.at[pl.ds(jbase, G)]], gbs[bset],
                         sems[bset])

    def wait_gather(bset, jbase):
        pltpu.make_async_copy(xT.at[colv.at[pl.ds(jbase, G)]], gbs[bset],
                              sems[bset]).wait()

    def pass_body(p, _):
        blk = p * (nc * ns) + wid
        base_row = blk * RB
        bvec = bndv[pl.ds(blk, L)]
        bs = bvec[0]
        be = bvec[1]

        z16 = jnp.zeros((L,), jnp.float32)

        def z_body(r, _):
            for t in range(B // L):
                acc[r, pl.ds(t * L, L)] = z16
            return 0
        lax.fori_loop(0, RB, z_body, 0)

        k0base = (bs // 8) * 8
        total = be - k0base
        nbatches = (total + BK - 1) // BK

        def batch_body(bi, _):
            bk0 = k0base + bi * BK
            pltpu.sync_copy(cols.at[pl.ds(bk0, BK)], colv.at[pl.ds(0, BK)])
            pltpu.sync_copy(rows.at[pl.ds(bk0, BK)], rowv)
            pltpu.sync_copy(vals.at[pl.ds(bk0, BK)], valv.at[pl.ds(0, BK)])
            for g in range(BK // L):
                kv = bk0 + g * L + lax.iota(jnp.int32, L)
                inr = (kv >= bs) & (kv < be)
                v = valv[pl.ds(g * L, L)]
                valv[pl.ds(g * L, L)] = jnp.where(
                    inr, v, jnp.zeros((L,), jnp.float32))
                r = rowv[pl.ds(g * L, L)]
                offv[pl.ds(g * L, L)] = jnp.clip(
                    r - base_row, 0, RB - 1)
            def vb_body(j, _):
                vj = valv[pl.ds(j, L)][0]
                vbuf[pl.ds(j * L, L)] = jnp.full((L,), vj, jnp.float32)
                return 0
            lax.fori_loop(0, BK, vb_body, 0)

            nchunks = jnp.minimum((be - bk0 + G - 1) // G, BCH)

            @pl.when(nchunks > 0)
            def _prologue():
                start_gather(0, 0)

            def process(ci, bset):
                @pl.when(ci + 1 < nchunks)
                def _next():
                    start_gather(1 - bset, (ci + 1) * G)
                wait_gather(bset, ci * G)
                nseg = _seg_scan(offv, segb, ci * G)
                for sp in range(NSTRIP):
                    _seg_accum(gbs[bset], acc, vbuf, offv, segb,
                               ci * G, nseg, sp)

            def pair_body(k, _):
                ci0 = 2 * k
                process(ci0, 0)

                @pl.when(ci0 + 1 < nchunks)
                def _odd():
                    process(ci0 + 1, 1)
                return 0
            lax.fori_loop(0, (nchunks + 1) // 2, pair_body, 0)
            return 0
        lax.fori_loop(0, nbatches, batch_body, 0)

        pltpu.sync_copy(acc, out.at[pl.ds(base_row, RB)])
        return 0

    lax.fori_loop(0, NPASS, pass_body, 0)


@jax.jit
def _spmm(xT, vals, rows, cols, bounds):
    mesh = plsc.VectorSubcoreMesh(core_axis_name="c", subcore_axis_name="s")
    kfn = functools.partial(
        pl.kernel,
        mesh=mesh,
        out_type=jax.ShapeDtypeStruct((N, B), jnp.float32),
        scratch_types=[
            pltpu.VMEM((G, B // 2), jnp.int32),
            pltpu.VMEM((G, B // 2), jnp.int32),
            pltpu.VMEM((RB, B), jnp.float32),
            pltpu.VMEM((BK + 16,), jnp.int32),
            pltpu.VMEM((BK,), jnp.int32),
            pltpu.VMEM((BK + 16,), jnp.float32),
            pltpu.VMEM((BK + 16,), jnp.int32),
            pltpu.VMEM((BK * L,), jnp.float32),
            pltpu.VMEM((BND_PAD,), jnp.int32),
            pltpu.SMEM((G + 8,), jnp.int32),
            pltpu.SemaphoreType.DMA,
            pltpu.SemaphoreType.DMA,
        ],
    )(_spmm_kernel)
    return kfn(xT, vals, rows, cols, bounds)


def kernel(x_affine, W_vals, W_row, W_col):
    xT = x_affine.T
    xTb = (xT.astype(jnp.bfloat16)
           .reshape(N, B // 32, 2, L)
           .transpose(0, 1, 3, 2)
           .reshape(N, B // 2, 2))
    xTb = lax.bitcast_convert_type(xTb, jnp.int32)
    rows = W_row.astype(jnp.int32)
    cols = W_col.astype(jnp.int32)
    rows_s, cols_s, vals_s = lax.sort((rows, cols, W_vals), num_keys=1)
    nnz = rows_s.shape[0]
    npad = ((nnz + 512 + 64) // 8) * 8
    pad = npad - nnz
    rows_s = jnp.concatenate([rows_s, jnp.full((pad,), N - 1, jnp.int32)])
    cols_s = jnp.concatenate([cols_s, jnp.zeros((pad,), jnp.int32)])
    vals_s = jnp.concatenate([vals_s, jnp.zeros((pad,), jnp.float32)])
    edges = jnp.arange(0, N + 1, RB, dtype=jnp.int32)
    bounds = jnp.searchsorted(rows_s[:nnz], edges).astype(jnp.int32)
    bounds = jnp.concatenate(
        [bounds, jnp.zeros((BND_PAD - NUM_BLOCKS - 1,), jnp.int32)])
    out_T = _spmm(xTb, vals_s, rows_s, cols_s, bounds)
    return out_T.T


# speedup vs baseline: 9.0425x; 1.1366x over previous
import functools

import jax
import jax.numpy as jnp
from jax import lax
from jax.experimental import pallas as pl
from jax.experimental.pallas import tpu as pltpu
from jax.experimental.pallas import tpu_sc as plsc

N = 16384
B = 1024
G = 32
RB = 64
NUM_BLOCKS = N // RB
NPASS = NUM_BLOCKS // 32
BND_PAD = NUM_BLOCKS + 1 + 15
L = 16
NSTRIP = 2
SCOLS = B // NSTRIP
SWORDS = SCOLS // 32
BK = 512
BCH = BK // G


def _seg_scan(offv, segb, jbase):
    def sc_body(j, carry):
        cnt, prev = carry
        off = offv[pl.ds(jbase + j, L)][0]
        changed = off != prev

        @pl.when(changed)
        def _mark():
            segb[cnt] = j
        return (cnt + jnp.where(changed, 1, 0), off)

    cnt, _ = lax.fori_loop(0, G, sc_body, (0, jnp.int32(-1)))
    segb[cnt] = G
    return cnt


def _seg_accum(gath, acc, vbuf, offv, segb, jbase, nseg, sp):
    zero16 = jnp.zeros((L,), jnp.float32)
    regs0 = (zero16,) * (2 * SWORDS)

    def s_body(si, _):
        jstart = segb[si]
        jend = segb[si + 1]
        off = offv[pl.ds(jbase + jstart, L)][0]

        def j_body(j, regs):
            vj = vbuf[pl.ds((jbase + j) * L, L)]
            new = []
            for w in range(SWORDS):
                wv = gath[j, pl.ds(sp * (SCOLS // 2) + w * L, L)]
                f_lo = lax.bitcast_convert_type(wv << 16, jnp.float32)
                f_hi = lax.bitcast_convert_type(
                    wv & jnp.int32(-65536), jnp.float32)
                new.append(regs[2 * w] + f_lo * vj)
                new.append(regs[2 * w + 1] + f_hi * vj)
            return tuple(new)

        regs = lax.fori_loop(jstart, jend, j_body, regs0)
        for w in range(SWORDS):
            cbase = sp * SCOLS + w * 32
            acc[off, pl.ds(cbase, L)] = (
                acc[off, pl.ds(cbase, L)] + regs[2 * w])
            acc[off, pl.ds(cbase + L, L)] = (
                acc[off, pl.ds(cbase + L, L)] + regs[2 * w + 1])
        return 0

    lax.fori_loop(0, nseg, s_body, 0)


def _spmm_kernel(xT, vals, rows, cols, bounds, out, gb0, gb1, acc, colv, rowv,
                 valv, offv, vbuf, bndv, segb, sem0, sem1):
    nc = 2
    ns = 16
    c = lax.axis_index("c")
    s = lax.axis_index("s")
    wid = s * nc + c

    gbs = (gb0, gb1)
    sems = (sem0, sem1)

    pltpu.sync_copy(bounds, bndv)

    def start_gather(bset, jbase):
        pltpu.async_copy(xT.at[colv.at[pl.ds(jbase, G)]], gbs[bset],
                         sems[bset])

    def wait_gather(bset, jbase):
        pltpu.make_async_copy(xT.at[colv.at[pl.ds(jbase, G)]], gbs[bset],
                              sems[bset]).wait()

    def pass_body(p, _):
        blk = p * (nc * ns) + wid
        base_row = blk * RB
        bvec = bndv[pl.ds(blk, L)]
        bs = bvec[0]
        be = bvec[1]

        z16 = jnp.zeros((L,), jnp.float32)

        def z_body(r, _):
            for t in range(B // L):
                acc[r, pl.ds(t * L, L)] = z16
            return 0
        lax.fori_loop(0, RB, z_body, 0)

        k0base = (bs // 8) * 8
        total = be - k0base
        nbatches = (total + BK - 1) // BK

        def batch_body(bi, _):
            bk0 = k0base + bi * BK
            pltpu.sync_copy(cols.at[pl.ds(bk0, BK)], colv.at[pl.ds(0, BK)])
            pltpu.sync_copy(rows.at[pl.ds(bk0, BK)], rowv)
            pltpu.sync_copy(vals.at[pl.ds(bk0, BK)], valv.at[pl.ds(0, BK)])
            for g in range(BK // L):
                kv = bk0 + g * L + lax.iota(jnp.int32, L)
                inr = (kv >= bs) & (kv < be)
                v = valv[pl.ds(g * L, L)]
                valv[pl.ds(g * L, L)] = jnp.where(
                    inr, v, jnp.zeros((L,), jnp.float32))
                r = rowv[pl.ds(g * L, L)]
                offv[pl.ds(g * L, L)] = jnp.clip(
                    r - base_row, 0, RB - 1)
            def vb_body(j, _):
                vj = valv[pl.ds(j, L)][0]
                vbuf[pl.ds(j * L, L)] = jnp.full((L,), vj, jnp.float32)
                return 0
            lax.fori_loop(0, BK, vb_body, 0)

            nchunks = jnp.minimum((be - bk0 + G - 1) // G, BCH)

            @pl.when(nchunks > 0)
            def _prologue():
                start_gather(0, 0)

            def process(ci, bset):
                @pl.when(ci + 1 < nchunks)
                def _next():
                    start_gather(1 - bset, (ci + 1) * G)
                wait_gather(bset, ci * G)
                nseg = _seg_scan(offv, segb, ci * G)
                for sp in range(NSTRIP):
                    _seg_accum(gbs[bset], acc, vbuf, offv, segb,
                               ci * G, nseg, sp)

            def pair_body(k, _):
                ci0 = 2 * k
                process(ci0, 0)

                @pl.when(ci0 + 1 < nchunks)
                def _odd():
                    process(ci0 + 1, 1)
                return 0
            lax.fori_loop(0, (nchunks + 1) // 2, pair_body, 0)
            return 0
        lax.fori_loop(0, nbatches, batch_body, 0)

        pltpu.sync_copy(acc, out.at[pl.ds(base_row, RB)])
        return 0

    lax.fori_loop(0, NPASS, pass_body, 0)


@jax.jit
def _spmm(xT, vals, rows, cols, bounds):
    mesh = plsc.VectorSubcoreMesh(core_axis_name="c", subcore_axis_name="s")
    kfn = functools.partial(
        pl.kernel,
        mesh=mesh,
        out_type=jax.ShapeDtypeStruct((N, B), jnp.float32),
        scratch_types=[
            pltpu.VMEM((G, B // 2), jnp.int32),
            pltpu.VMEM((G, B // 2), jnp.int32),
            pltpu.VMEM((RB, B), jnp.float32),
            pltpu.VMEM((BK + 16,), jnp.int32),
            pltpu.VMEM((BK,), jnp.int32),
            pltpu.VMEM((BK + 16,), jnp.float32),
            pltpu.VMEM((BK + 16,), jnp.int32),
            pltpu.VMEM((BK * L,), jnp.float32),
            pltpu.VMEM((BND_PAD,), jnp.int32),
            pltpu.SMEM((G + 8,), jnp.int32),
            pltpu.SemaphoreType.DMA,
            pltpu.SemaphoreType.DMA,
        ],
    )(_spmm_kernel)
    return kfn(xT, vals, rows, cols, bounds)


def kernel(x_affine, W_vals, W_row, W_col):
    xT = x_affine.T
    xTb = (xT.astype(jnp.bfloat16)
           .reshape(N, B // 32, 2, L)
           .transpose(0, 1, 3, 2)
           .reshape(N, B // 2, 2))
    xTb = lax.bitcast_convert_type(xTb, jnp.int32)
    rows = W_row.astype(jnp.int32)
    cols = W_col.astype(jnp.int32)
    rows_s, cols_s, vals_s = lax.sort((rows, cols, W_vals), num_keys=1, is_stable=False)
    nnz = rows_s.shape[0]
    npad = ((nnz + 512 + 64) // 8) * 8
    pad = npad - nnz
    rows_s = jnp.concatenate([rows_s, jnp.full((pad,), N - 1, jnp.int32)])
    cols_s = jnp.concatenate([cols_s, jnp.zeros((pad,), jnp.int32)])
    vals_s = jnp.concatenate([vals_s, jnp.zeros((pad,), jnp.float32)])
    edges = jnp.arange(0, N + 1, RB, dtype=jnp.int32)
    bounds = jnp.searchsorted(rows_s[:nnz], edges).astype(jnp.int32)
    bounds = jnp.concatenate(
        [bounds, jnp.zeros((BND_PAD - NUM_BLOCKS - 1,), jnp.int32)])
    out_T = _spmm(xTb, vals_s, rows_s, cols_s, bounds)
    return out_T.T
